# Initial kernel scaffold; baseline (speedup 1.0000x reference)
import functools

import jax
import jax.numpy as jnp
from jax import lax
from jax.experimental import pallas as pl
from jax.experimental.pallas import tpu as pltpu
from jax.experimental.pallas import tpu_sc as plsc

B, N, K, D = 4096, 200, 50, 128
NC, NS, L = 2, 16, 16
NW = NC * NS
ROWS = B * K
RPW = ROWS // NW
CHUNK = 128
NCHUNK = RPW // CHUNK

_mesh = plsc.VectorSubcoreMesh(
    core_axis_name="c", subcore_axis_name="s", num_cores=NC, num_subcores=NS
)


@functools.partial(
    pl.kernel,
    out_type=jax.ShapeDtypeStruct((ROWS, D), jnp.float32),
    mesh=_mesh,
    scratch_types=[
        pltpu.VMEM((CHUNK,), jnp.int32),
        pltpu.VMEM((CHUNK, D), jnp.float32),
        pltpu.SemaphoreType.DMA,
    ],
)
def _gather(src_hbm, idx_hbm, out_hbm, idx_v, rows_v, sem):
    wid = lax.axis_index("s") * NC + lax.axis_index("c")
    wbase = wid * RPW
    lane = lax.iota(jnp.int32, 16)

    def chunk_body(c, carry):
        base = wbase + c * CHUNK
        pltpu.sync_copy(idx_hbm.at[pl.ds(base, CHUNK)], idx_v)
        for i in range(CHUNK // L):
            r = base + i * L + lane
            idx_v[pl.ds(i * L, L)] = (r // K) * N + idx_v[pl.ds(i * L, L)]
        pltpu.async_copy(src_hbm.at[idx_v], rows_v, sem).wait()
        pltpu.sync_copy(rows_v, out_hbm.at[pl.ds(base, CHUNK)])
        return carry

    lax.fori_loop(0, NCHUNK, chunk_body, 0)


def kernel(source, idxs):
    src = source.reshape(B * N, D)
    idx = idxs.astype(jnp.int32).reshape(ROWS)
    out = _gather(src, idx)
    return out.reshape(B, K, D)


# speedup vs baseline: 1.0573x; 1.0573x over previous
import functools

import jax
import jax.numpy as jnp
from jax import lax
from jax.experimental import pallas as pl
from jax.experimental.pallas import tpu as pltpu
from jax.experimental.pallas import tpu_sc as plsc

B, N, K, D = 4096, 200, 50, 128
NC, NS, L = 2, 16, 16
NW = NC * NS
ROWS = B * K
RPW = ROWS // NW
CHUNK = 128
NCHUNK = RPW // CHUNK

_mesh = plsc.VectorSubcoreMesh(
    core_axis_name="c", subcore_axis_name="s", num_cores=NC, num_subcores=NS
)


_MAGIC = 41944
_SHIFT = 21


@functools.partial(
    pl.kernel,
    out_type=jax.ShapeDtypeStruct((ROWS, D), jnp.float32),
    mesh=_mesh,
    scratch_types=[
        pltpu.VMEM((CHUNK,), jnp.int32),
        pltpu.VMEM((CHUNK, D), jnp.float32),
        pltpu.SemaphoreType.DMA,
    ],
)
def _gather(src_hbm, idx_hbm, out_hbm, idx_v, rows_v, sem):
    wid = lax.axis_index("s") * NC + lax.axis_index("c")
    wbase = wid * RPW
    bbase = wid * (RPW // K)
    lane = lax.iota(jnp.int32, L)

    def chunk_body(c, carry):
        base = wbase + c * CHUNK
        pltpu.sync_copy(idx_hbm.at[pl.ds(base, CHUNK)], idx_v)
        for i in range(CHUNK // L):
            t = c * CHUNK + i * L + lane
            b = bbase + lax.shift_right_logical(t * _MAGIC, _SHIFT)
            idx_v[pl.ds(i * L, L)] = b * N + idx_v[pl.ds(i * L, L)]
        pltpu.async_copy(src_hbm.at[idx_v], rows_v, sem).wait()
        pltpu.sync_copy(rows_v, out_hbm.at[pl.ds(base, CHUNK)])
        return carry

    lax.fori_loop(0, NCHUNK, chunk_body, 0)


def kernel(source, idxs):
    src = source.reshape(B * N, D)
    idx = idxs.astype(jnp.int32).reshape(ROWS)
    out = _gather(src, idx)
    return out.reshape(B, K, D)


# speedup vs baseline: 1.2811x; 1.2116x over previous
import functools

import jax
import jax.numpy as jnp
from jax import lax
from jax.experimental import pallas as pl
from jax.experimental.pallas import tpu as pltpu
from jax.experimental.pallas import tpu_sc as plsc

B, N, K, D = 4096, 200, 50, 128
NC, NS, L = 2, 16, 16
NW = NC * NS
ROWS = B * K
RPW = ROWS // NW
CHUNK = 128
NCHUNK = RPW // CHUNK
NB = 4

_MAGIC = 41944
_SHIFT = 21

_mesh = plsc.VectorSubcoreMesh(
    core_axis_name="c", subcore_axis_name="s", num_cores=NC, num_subcores=NS
)


@functools.partial(
    pl.kernel,
    out_type=jax.ShapeDtypeStruct((ROWS, D), jnp.float32),
    mesh=_mesh,
    scratch_types=[
        pltpu.VMEM((RPW,), jnp.int32),
        pltpu.VMEM((NB, CHUNK, D), jnp.float32),
    ]
    + [pltpu.SemaphoreType.DMA] * (2 * NB),
)
def _gather(src_hbm, idx_hbm, out_hbm, idx_all, rows, *sems):
    sem_g = sems[:NB]
    sem_o = sems[NB:]
    wid = lax.axis_index("s") * NC + lax.axis_index("c")
    wbase = wid * RPW
    bbase = wid * (RPW // K)
    lane = lax.iota(jnp.int32, L)

    pltpu.sync_copy(idx_hbm.at[pl.ds(wbase, RPW)], idx_all)

    def flatten_chunk(g):
        for i in range(CHUNK // L):
            off = g * CHUNK + i * L
            b = bbase + lax.shift_right_logical((off + lane) * _MAGIC, _SHIFT)
            idx_all[pl.ds(off, L)] = b * N + idx_all[pl.ds(off, L)]

    gathers = {}
    writes = {}

    def start_gather(g):
        gathers[g] = pltpu.async_copy(
            src_hbm.at[idx_all.at[pl.ds(g * CHUNK, CHUNK)]],
            rows.at[g % NB],
            sem_g[g % NB],
        )

    def start_write(g):
        writes[g] = pltpu.async_copy(
            rows.at[g % NB],
            out_hbm.at[pl.ds(wbase + g * CHUNK, CHUNK)],
            sem_o[g % NB],
        )

    for g in range(NCHUNK):
        flatten_chunk(g)
        if g >= NB:
            writes[g - NB].wait()
        start_gather(g)
        if g >= NB - 1:
            gathers[g - (NB - 1)].wait()
            start_write(g - (NB - 1))
    for g in range(NCHUNK - (NB - 1), NCHUNK):
        gathers[g].wait()
        start_write(g)
    for g in range(NCHUNK - NB, NCHUNK):
        writes[g].wait()


def kernel(source, idxs):
    src = source.reshape(B * N, D)
    idx = idxs.astype(jnp.int32).reshape(ROWS)
    out = _gather(src, idx)
    return out.reshape(B, K, D)


# speedup vs baseline: 1.2848x; 1.0029x over previous
import functools

import jax
import jax.numpy as jnp
from jax import lax
from jax.experimental import pallas as pl
from jax.experimental.pallas import tpu as pltpu
from jax.experimental.pallas import tpu_sc as plsc

B, N, K, D = 4096, 200, 50, 128
NC, NS, L = 2, 16, 16
NW = NC * NS
ROWS = B * K
RPW = ROWS // NW
CHUNK = 128
NCHUNK = RPW // CHUNK
NB = 6

_MAGIC = 41944
_SHIFT = 21

_mesh = plsc.VectorSubcoreMesh(
    core_axis_name="c", subcore_axis_name="s", num_cores=NC, num_subcores=NS
)


@functools.partial(
    pl.kernel,
    out_type=jax.ShapeDtypeStruct((ROWS, D), jnp.float32),
    mesh=_mesh,
    scratch_types=[
        pltpu.VMEM((RPW,), jnp.int32),
        pltpu.VMEM((NB, CHUNK, D), jnp.float32),
    ]
    + [pltpu.SemaphoreType.DMA] * (2 * NB),
)
def _gather(src_hbm, idx_hbm, out_hbm, idx_all, rows, *sems):
    sem_g = sems[:NB]
    sem_o = sems[NB:]
    wid = lax.axis_index("s") * NC + lax.axis_index("c")
    wbase = wid * RPW
    bbase = wid * (RPW // K)
    lane = lax.iota(jnp.int32, L)

    pltpu.sync_copy(idx_hbm.at[pl.ds(wbase, RPW)], idx_all)

    def flatten_chunk(g):
        for i in range(CHUNK // L):
            off = g * CHUNK + i * L
            b = bbase + lax.shift_right_logical((off + lane) * _MAGIC, _SHIFT)
            idx_all[pl.ds(off, L)] = b * N + idx_all[pl.ds(off, L)]

    gathers = {}
    writes = {}

    def start_gather(g):
        gathers[g] = pltpu.async_copy(
            src_hbm.at[idx_all.at[pl.ds(g * CHUNK, CHUNK)]],
            rows.at[g % NB],
            sem_g[g % NB],
        )

    def start_write(g):
        writes[g] = pltpu.async_copy(
            rows.at[g % NB],
            out_hbm.at[pl.ds(wbase + g * CHUNK, CHUNK)],
            sem_o[g % NB],
        )

    for g in range(NCHUNK):
        flatten_chunk(g)
        if g >= NB:
            writes[g - NB].wait()
        start_gather(g)
        if g >= NB - 1:
            gathers[g - (NB - 1)].wait()
            start_write(g - (NB - 1))
    for g in range(NCHUNK - (NB - 1), NCHUNK):
        gathers[g].wait()
        start_write(g)
    for g in range(NCHUNK - NB, NCHUNK):
        writes[g].wait()


def kernel(source, idxs):
    src = source.reshape(B * N, D)
    idx = idxs.astype(jnp.int32).reshape(ROWS)
    out = _gather(src, idx)
    return out.reshape(B, K, D)


# speedup vs baseline: 1.2893x; 1.0034x over previous
import functools

import jax
import jax.numpy as jnp
from jax import lax
from jax.experimental import pallas as pl
from jax.experimental.pallas import tpu as pltpu
from jax.experimental.pallas import tpu_sc as plsc

B, N, K, D = 4096, 200, 50, 128
NC, NS, L = 2, 16, 16
NW = NC * NS
ROWS = B * K
RPW = ROWS // NW
CHUNK = 128
NCHUNK = RPW // CHUNK
NB = 6

_MAGIC = 41944
_SHIFT = 21

_mesh = plsc.VectorSubcoreMesh(
    core_axis_name="c", subcore_axis_name="s", num_cores=NC, num_subcores=NS
)


@functools.partial(
    pl.kernel,
    out_type=jax.ShapeDtypeStruct((ROWS, D), jnp.float32),
    mesh=_mesh,
    scratch_types=[
        pltpu.VMEM((RPW,), jnp.int32),
        pltpu.VMEM((NB, CHUNK, D), jnp.float32),
    ]
    + [pltpu.SemaphoreType.DMA] * (2 * NB),
    compiler_params=pltpu.CompilerParams(use_tc_tiling_on_sc=True),
)
def _gather(src_hbm, idx_hbm, out_hbm, idx_all, rows, *sems):
    sem_g = sems[:NB]
    sem_o = sems[NB:]
    wid = lax.axis_index("s") * NC + lax.axis_index("c")
    wbase = wid * RPW
    bbase = wid * (RPW // K)
    lane = lax.iota(jnp.int32, L)

    pltpu.sync_copy(idx_hbm.at[pl.ds(wbase, RPW)], idx_all)

    def flatten_chunk(g):
        for i in range(CHUNK // L):
            off = g * CHUNK + i * L
            b = bbase + lax.shift_right_logical((off + lane) * _MAGIC, _SHIFT)
            idx_all[pl.ds(off, L)] = b * N + idx_all[pl.ds(off, L)]

    gathers = {}
    writes = {}

    def start_gather(g):
        gathers[g] = pltpu.async_copy(
            src_hbm.at[idx_all.at[pl.ds(g * CHUNK, CHUNK)]],
            rows.at[g % NB],
            sem_g[g % NB],
        )

    def start_write(g):
        writes[g] = pltpu.async_copy(
            rows.at[g % NB],
            out_hbm.at[pl.ds(wbase + g * CHUNK, CHUNK)],
            sem_o[g % NB],
        )

    for g in range(NCHUNK):
        flatten_chunk(g)
        if g >= NB:
            writes[g - NB].wait()
        start_gather(g)
        if g >= NB - 1:
            gathers[g - (NB - 1)].wait()
            start_write(g - (NB - 1))
    for g in range(NCHUNK - (NB - 1), NCHUNK):
        gathers[g].wait()
        start_write(g)
    for g in range(NCHUNK - NB, NCHUNK):
        writes[g].wait()


def kernel(source, idxs):
    src = source.reshape(B * N, D)
    idx = idxs.astype(jnp.int32).reshape(ROWS)
    out = _gather(src, idx)
    return out.reshape(B, K, D)


# speedup vs baseline: 1.8559x; 1.4395x over previous
import functools

import jax
import jax.numpy as jnp
from jax import lax
from jax.experimental import pallas as pl
from jax.experimental.pallas import tpu as pltpu
from jax.experimental.pallas import tpu_sc as plsc

B, N, K, D = 4096, 200, 50, 128
PADK = 56
NC, NS, L = 2, 16, 16
NW = NC * NS
ROWS = B * K
BPW = B // NW
RPW = ROWS // NW
PPW = BPW * PADK
BPC = 2
CHUNK = BPC * PADK
NCHUNK = BPW // BPC
NB = 6

_MAGIC = 9363
_SHIFT = 16

_mesh = plsc.VectorSubcoreMesh(
    core_axis_name="c", subcore_axis_name="s", num_cores=NC, num_subcores=NS
)


@functools.partial(
    pl.kernel,
    out_type=jax.ShapeDtypeStruct((B * PADK, D), jnp.float32),
    mesh=_mesh,
    scratch_types=[
        pltpu.VMEM((RPW,), jnp.int32),
        pltpu.VMEM((PPW,), jnp.int32),
        pltpu.VMEM((NB, CHUNK, D), jnp.float32),
    ]
    + [pltpu.SemaphoreType.DMA] * (2 * NB),
    compiler_params=pltpu.CompilerParams(needs_layout_passes=False),
)
def _gather(src_hbm, idx_hbm, out_hbm, idx_raw, idx_pad, rows, *sems):
    sem_g = sems[:NB]
    sem_o = sems[NB:]
    wid = lax.axis_index("s") * NC + lax.axis_index("c")
    wbase = wid * RPW
    bbase = wid * BPW
    lane = lax.iota(jnp.int32, L)

    pltpu.sync_copy(idx_hbm.at[pl.ds(wbase, RPW)], idx_raw)

    def build_pad_chunk(g):
        for u in range(CHUNK // L):
            p = g * CHUNK + u * L + lane
            j = lax.shift_right_logical(
                lax.shift_right_logical(p, 3) * _MAGIC, _SHIFT
            )
            k = p - j * PADK
            t = j * K + jnp.minimum(k, K - 1)
            raw = plsc.load_gather(idx_raw, [t])
            idx_pad[pl.ds(g * CHUNK + u * L, L)] = (bbase + j) * N + raw

    gathers = {}
    writes = {}

    def start_gather(g):
        gathers[g] = pltpu.async_copy(
            src_hbm.at[idx_pad.at[pl.ds(g * CHUNK, CHUNK)]],
            rows.at[g % NB],
            sem_g[g % NB],
        )

    def start_write(g):
        writes[g] = pltpu.async_copy(
            rows.at[g % NB],
            out_hbm.at[pl.ds((bbase + g * BPC) * PADK, CHUNK)],
            sem_o[g % NB],
        )

    for g in range(NCHUNK):
        build_pad_chunk(g)
        if g >= NB:
            writes[g - NB].wait()
        start_gather(g)
        if g >= NB - 1:
            gathers[g - (NB - 1)].wait()
            start_write(g - (NB - 1))
    for g in range(NCHUNK - (NB - 1), NCHUNK):
        gathers[g].wait()
        start_write(g)
    for g in range(NCHUNK - NB, NCHUNK):
        writes[g].wait()


def kernel(source, idxs):
    src = source.reshape(B * N, D)
    idx = idxs.astype(jnp.int32).reshape(ROWS)
    out = _gather(src, idx)
    return out.reshape(B, PADK, D)[:, :K, :]
